# baseline (speedup 1.0000x reference)
import functools

import jax
import jax.numpy as jnp
from jax import lax
from jax.experimental import pallas as pl
from jax.experimental.pallas import tpu as pltpu
from jax.experimental.pallas import tpu_sc as plsc

N = 10000
E = 320000
F = 128
G = 16
NC, NS, L = 2, 16, 16
NW = NC * NS
EPW = E // NW
NPAD = 10112
RPT = NPAD // NS
DR = 80
CH = 32
NFULL = 312
TAIL = EPW - NFULL * CH


def _prep_body(x1_ref, W1_ref, as1_ref, ad1_ref, x2_ref, W2_ref, as2_ref,
               ad2_ref, h1_ref, avec1_ref, h2_ref, avec2_ref):
    for x_ref, W_ref, asr, adr, h_ref, avec_ref in (
        (x1_ref, W1_ref, as1_ref, ad1_ref, h1_ref, avec1_ref),
        (x2_ref, W2_ref, as2_ref, ad2_ref, h2_ref, avec2_ref),
    ):
        h = jnp.dot(x_ref[...], W_ref[...], preferred_element_type=jnp.float32)
        h_ref[...] = h
        a_s = jnp.sum(h * asr[...][None, :], axis=1)
        a_d = jnp.sum(h * adr[...][None, :], axis=1)
        avec_ref[...] = jnp.stack([a_s, a_d])


_prep = pl.pallas_call(
    _prep_body,
    out_shape=[
        jax.ShapeDtypeStruct((N, F), jnp.float32),
        jax.ShapeDtypeStruct((2, N), jnp.float32),
        jax.ShapeDtypeStruct((N, F), jnp.float32),
        jax.ShapeDtypeStruct((2, N), jnp.float32),
    ],
)


def _sc_edge_body(src_ref, dst_ref, h_ref, as_ref, ad_ref,
                  acc_out, den_out,
                  acc_sh, asl, adl, denv, exv,
                  srcv0, srcv1, srcv2, srcv3, srcv4, srcv5, srcv6, srcv7,
                  dstv0, dstv1, dstv2, dstv3, dstv4, dstv5, dstv6, dstv7,
                  rows0, rows1, rows2, rows3,
                  semg0, semg1, semg2, semg3, sems0, sems1, sems2, sems3,
                  semi0, semi1, semi2, semi3, semi4, semi5, semi6, semi7):
    cid = lax.axis_index("c")
    sid = lax.axis_index("s")
    wid = cid * NS + sid

    def zden(r, carry):
        for c in range(F // L):
            denv[r, pl.ds(c * L, L)] = jnp.zeros((L,), jnp.float32)
        return carry
    lax.fori_loop(0, DR, zden, 0)
    for i in range(RPT // DR):
        pltpu.sync_copy(denv, acc_sh.at[pl.ds(sid * RPT + i * DR, DR)])
    rem = RPT % DR
    if rem:
        pltpu.sync_copy(
            denv.at[pl.ds(0, rem)],
            acc_sh.at[pl.ds(sid * RPT + (RPT // DR) * DR, rem)])

    pltpu.sync_copy(as_ref, asl)
    pltpu.sync_copy(ad_ref, adl)

    plsc.subcore_barrier()

    ebase = wid * EPW
    srcs = [srcv0, srcv1, srcv2, srcv3, srcv4, srcv5, srcv6, srcv7]
    dsts = [dstv0, dstv1, dstv2, dstv3, dstv4, dstv5, dstv6, dstv7]
    rowss = [rows0, rows1, rows2, rows3]
    semgs = [semg0, semg1, semg2, semg3]
    semss = [sems0, sems1, sems2, sems3]
    semis = [semi0, semi1, semi2, semi3, semi4, semi5, semi6, semi7]

    def idx_start(k, q):
        base = ebase + k * CH
        pltpu.async_copy(src_ref.at[pl.ds(base, CH)], srcs[q], semis[q])
        pltpu.async_copy(dst_ref.at[pl.ds(base, CH)], dsts[q], semis[q])

    def idx_wait(k, q):
        base = ebase + k * CH
        pltpu.make_async_copy(src_ref.at[pl.ds(base, CH)], srcs[q],
                              semis[q]).wait()
        pltpu.make_async_copy(dst_ref.at[pl.ds(base, CH)], dsts[q],
                              semis[q]).wait()

    def gather_start(s, q):
        pltpu.async_copy(h_ref.at[srcs[q]], rowss[s], semgs[s])

    def gather_wait(s, q):
        pltpu.make_async_copy(h_ref.at[srcs[q]], rowss[s], semgs[s]).wait()

    def scatter_start(s, q):
        pltpu.async_copy(rowss[s], acc_sh.at[dsts[q]], semss[s], add=True)

    def scatter_wait(s, q):
        pltpu.make_async_copy(rowss[s], acc_sh.at[dsts[q]], semss[s]).wait()

    def scores(q):
        for g in range(CH // L):
            si = srcs[q][pl.ds(g * L, L)]
            di = dsts[q][pl.ds(g * L, L)]
            e = plsc.load_gather(asl, [si]) + plsc.load_gather(adl, [di])
            e = jnp.where(e >= 0, e, 0.2 * e)
            ex = jnp.exp(e)
            exv[pl.ds(g * L, L)] = ex
            plsc.addupdate_scatter(
                denv, [lax.shift_right_logical(di, 7),
                       lax.bitwise_and(di, 127)], ex)

    def scale(s):
        def srow(i, carry):
            for jj in range(16):
                j = i * 16 + jj
                sv = plsc.load_gather(exv, [jnp.full((L,), j, jnp.int32)])
                for c in range(F // L):
                    rowss[s][j, pl.ds(c * L, L)] = (
                        rowss[s][j, pl.ds(c * L, L)] * sv)
            return carry
        lax.fori_loop(0, CH // 16, srow, 0)

    for k in range(4):
        idx_start(k, k)
    idx_wait(0, 0)
    idx_wait(1, 1)
    gather_start(0, 0)
    gather_start(1, 1)

    NOCT = NFULL // 8

    def octet(k8, carry):
        for s8 in range(8):
            k = k8 * 8 + s8
            s = s8 % 4
            scores(s8)
            gather_wait(s, s8)
            scale(s)
            scatter_start(s, s8)
            s2 = (s + 2) % 4
            q2 = (s8 + 2) % 8
            q4 = (s8 + 4) % 8

            if s8 < 2:

                @pl.when(k8 > 0)
                def _():
                    scatter_wait(s2, q2)
            else:
                scatter_wait(s2, q2)

            if s8 < 4:
                idx_start(k + 4, q4)
                idx_wait(k + 2, q2)
                gather_start(s2, q2)
            elif s8 < 6:

                @pl.when(k8 < NOCT - 1)
                def _():
                    idx_start(k + 4, q4)

                idx_wait(k + 2, q2)
                gather_start(s2, q2)
            else:

                @pl.when(k8 < NOCT - 1)
                def _():
                    idx_start(k + 4, q4)
                    idx_wait(k + 2, q2)
                    gather_start(s2, q2)
        return carry

    lax.fori_loop(0, NOCT, octet, 0)

    scatter_wait(2, 6)
    scatter_wait(3, 7)

    tbase = ebase + NFULL * CH
    pltpu.sync_copy(src_ref.at[pl.ds(tbase, TAIL)], srcv0.at[pl.ds(0, TAIL)])
    pltpu.sync_copy(dst_ref.at[pl.ds(tbase, TAIL)], dstv0.at[pl.ds(0, TAIL)])
    srcv0[pl.ds(TAIL, L)] = jnp.zeros((L,), jnp.int32)
    dstv0[pl.ds(TAIL, L)] = jnp.full((L,), NPAD - 8, jnp.int32)
    scores(0)
    exv[pl.ds(TAIL, L)] = jnp.zeros((L,), jnp.float32)
    gather_start(0, 0)
    gather_wait(0, 0)
    scale(0)
    scatter_start(0, 0)
    scatter_wait(0, 0)

    plsc.subcore_barrier()

    pltpu.sync_copy(acc_sh.at[pl.ds(sid * RPT, RPT)],
                    acc_out.at[cid, pl.ds(sid * RPT, RPT)])
    pltpu.sync_copy(denv, den_out.at[wid])


@functools.lru_cache(maxsize=None)
def _get_sc_edge():
    return functools.partial(
        pl.kernel,
        out_type=[
            jax.ShapeDtypeStruct((NC, NPAD, F), jnp.float32),
            jax.ShapeDtypeStruct((NW, DR, F), jnp.float32),
        ],
        mesh=plsc.VectorSubcoreMesh(core_axis_name="c", subcore_axis_name="s",
                                    num_cores=NC, num_subcores=NS),
        compiler_params=pltpu.CompilerParams(needs_layout_passes=False),
        scratch_types=(
            [
                pltpu.VMEM_SHARED((NPAD, F), jnp.float32),
                pltpu.VMEM((N,), jnp.float32),
                pltpu.VMEM((N,), jnp.float32),
                pltpu.VMEM((DR, F), jnp.float32),
                pltpu.VMEM((CH,), jnp.float32),
            ]
            + [pltpu.VMEM((CH,), jnp.int32) for _ in range(16)]
            + [pltpu.VMEM((CH, F), jnp.float32) for _ in range(4)]
            + [pltpu.SemaphoreType.DMA for _ in range(16)]
        ),
    )(_sc_edge_body)


def _head_body(acc1_ref, den1_ref, acc2_ref, den2_ref, b1_ref, b2_ref,
               batch1_ref, batch2_ref,
               fcW1_ref, fcb1_ref, fcW2_ref, fcb2_ref,
               fcAW_ref, fcAb_ref, fcBW_ref, fcBb_ref, outW_ref, outb_ref,
               out_ref):
    def lr(z):
        return jnp.where(z >= 0, z, 0.01 * z)

    def gat_out(acc_ref, den_ref, b_ref):
        h = acc_ref[0, :N] + acc_ref[1, :N]
        den = jnp.sum(den_ref[...], axis=0)[:N]
        return h / (den[:, None] + 1e-16) + b_ref[...][None, :]

    def pool(x, batch_ref):
        b = batch_ref[...]
        seg = lax.broadcasted_iota(jnp.int32, (G, N), 0)
        P = jnp.where(seg == b[None, :], 1.0, 0.0)
        s = jnp.dot(P, x, preferred_element_type=jnp.float32)
        c = jnp.sum(P, axis=1, keepdims=True)
        return s / jnp.maximum(c, 1.0)

    x = lr(gat_out(acc1_ref, den1_ref, b1_ref))
    x = pool(x, batch1_ref)
    x = lr(jnp.dot(x, fcW1_ref[...], preferred_element_type=jnp.float32)
           + fcb1_ref[...][None, :])

    xt = gat_out(acc2_ref, den2_ref, b2_ref)
    xt = lr(jnp.dot(xt, fcW2_ref[...], preferred_element_type=jnp.float32)
            + fcb2_ref[...][None, :])
    xt = lr(pool(xt, batch2_ref))

    xc = jnp.concatenate([x, xt], axis=1)
    xc = lr(jnp.dot(xc, fcAW_ref[...], preferred_element_type=jnp.float32)
            + fcAb_ref[...][None, :])
    xc = lr(jnp.dot(xc, fcBW_ref[...], preferred_element_type=jnp.float32)
            + fcBb_ref[...][None, :])
    z = (jnp.dot(xc, outW_ref[...], preferred_element_type=jnp.float32)
         + outb_ref[...][None, :])
    out_ref[...] = 1.0 / (1.0 + jnp.exp(-z))


_head = pl.pallas_call(
    _head_body,
    out_shape=jax.ShapeDtypeStruct((G, F), jnp.float32),
)


def kernel(pro1_x, pro1_edge_index, pro1_batch, pro2_x, pro2_edge_index,
           pro2_batch, W1, asrc1, adst1, b1, fcW_p1, fcb_p1,
           W2, asrc2, adst2, b2, fcW_p2, fcb_p2,
           fcAW, fcAb, fcBW, fcBb, outW, outb):
    h1, avec1, h2, avec2 = _prep(
        pro1_x, W1, asrc1, adst1, pro2_x, W2, asrc2, adst2)
    sc_edge = _get_sc_edge()
    acc1, den1 = sc_edge(pro1_edge_index[0], pro1_edge_index[1], h1,
                         avec1[0], avec1[1])
    acc2, den2 = sc_edge(pro2_edge_index[0], pro2_edge_index[1], h2,
                         avec2[0], avec2[1])
    den1 = den1.reshape(NW, DR * F)
    den2 = den2.reshape(NW, DR * F)
    outWp = jnp.pad(outW, ((0, 0), (0, F - 1)))
    outbp = jnp.pad(outb, ((0, F - 1),))
    o = _head(acc1, den1, acc2, den2, b1, b2, pro1_batch, pro2_batch,
              fcW_p1, fcb_p1, fcW_p2, fcb_p2,
              fcAW, fcAb, fcBW, fcBb, outWp, outbp)
    return o[:, :1]


# speedup vs baseline: 1.0572x; 1.0572x over previous
import functools

import jax
import jax.numpy as jnp
from jax import lax
from jax.experimental import pallas as pl
from jax.experimental.pallas import tpu as pltpu
from jax.experimental.pallas import tpu_sc as plsc

N = 10000
E = 320000
F = 128
G = 16
NC, NS, L = 2, 16, 16
NW = NC * NS
EPW = E // NW
NPAD = 10112
RPT = NPAD // NS
DR = 80
CH = 32
NFULL = 312
TAIL = EPW - NFULL * CH


def _prep_body(x1_ref, W1_ref, as1_ref, ad1_ref, x2_ref, W2_ref, as2_ref,
               ad2_ref, h1_ref, avec1_ref, h2_ref, avec2_ref):
    for x_ref, W_ref, asr, adr, h_ref, avec_ref in (
        (x1_ref, W1_ref, as1_ref, ad1_ref, h1_ref, avec1_ref),
        (x2_ref, W2_ref, as2_ref, ad2_ref, h2_ref, avec2_ref),
    ):
        h = jnp.dot(x_ref[...], W_ref[...], preferred_element_type=jnp.float32)
        h_ref[...] = h
        a_s = jnp.sum(h * asr[...][None, :], axis=1)
        a_d = jnp.sum(h * adr[...][None, :], axis=1)
        avec_ref[...] = jnp.stack([a_s, a_d])


_prep = pl.pallas_call(
    _prep_body,
    out_shape=[
        jax.ShapeDtypeStruct((N, F), jnp.float32),
        jax.ShapeDtypeStruct((2, N), jnp.float32),
        jax.ShapeDtypeStruct((N, F), jnp.float32),
        jax.ShapeDtypeStruct((2, N), jnp.float32),
    ],
)


def _sc_edge_body(src1_ref, dst1_ref, h1_ref, as1_ref, ad1_ref,
                  src2_ref, dst2_ref, h2_ref, as2_ref, ad2_ref,
                  acc1_out, den1_out, acc2_out, den2_out,
                  acc_sh, asl, adl, denv, exv,
                  srcv0, srcv1, srcv2, srcv3, srcv4, srcv5, srcv6, srcv7,
                  dstv0, dstv1, dstv2, dstv3, dstv4, dstv5, dstv6, dstv7,
                  rows0, rows1, rows2, rows3,
                  semg0, semg1, semg2, semg3, sems0, sems1, sems2, sems3,
                  semi0, semi1, semi2, semi3, semi4, semi5, semi6, semi7):
    cid = lax.axis_index("c")
    sid = lax.axis_index("s")
    wid = cid * NS + sid

    def run_branch(src_ref, dst_ref, h_ref, as_ref, ad_ref, acc_out, den_out):
        def zden(r, carry):
            for c in range(F // L):
                denv[r, pl.ds(c * L, L)] = jnp.zeros((L,), jnp.float32)
            return carry
        lax.fori_loop(0, DR, zden, 0)
        for i in range(RPT // DR):
            pltpu.sync_copy(denv, acc_sh.at[pl.ds(sid * RPT + i * DR, DR)])
        rem = RPT % DR
        if rem:
            pltpu.sync_copy(
                denv.at[pl.ds(0, rem)],
                acc_sh.at[pl.ds(sid * RPT + (RPT // DR) * DR, rem)])

        pltpu.sync_copy(as_ref, asl)
        pltpu.sync_copy(ad_ref, adl)

        plsc.subcore_barrier()

        ebase = wid * EPW
        srcs = [srcv0, srcv1, srcv2, srcv3, srcv4, srcv5, srcv6, srcv7]
        dsts = [dstv0, dstv1, dstv2, dstv3, dstv4, dstv5, dstv6, dstv7]
        rowss = [rows0, rows1, rows2, rows3]
        semgs = [semg0, semg1, semg2, semg3]
        semss = [sems0, sems1, sems2, sems3]
        semis = [semi0, semi1, semi2, semi3, semi4, semi5, semi6, semi7]

        def idx_start(k, q):
            base = ebase + k * CH
            pltpu.async_copy(src_ref.at[pl.ds(base, CH)], srcs[q], semis[q])
            pltpu.async_copy(dst_ref.at[pl.ds(base, CH)], dsts[q], semis[q])

        def idx_wait(k, q):
            base = ebase + k * CH
            pltpu.make_async_copy(src_ref.at[pl.ds(base, CH)], srcs[q],
                                  semis[q]).wait()
            pltpu.make_async_copy(dst_ref.at[pl.ds(base, CH)], dsts[q],
                                  semis[q]).wait()

        def gather_start(s, q):
            pltpu.async_copy(h_ref.at[srcs[q]], rowss[s], semgs[s])

        def gather_wait(s, q):
            pltpu.make_async_copy(h_ref.at[srcs[q]], rowss[s], semgs[s]).wait()

        def scatter_start(s, q):
            pltpu.async_copy(rowss[s], acc_sh.at[dsts[q]], semss[s], add=True)

        def scatter_wait(s, q):
            pltpu.make_async_copy(rowss[s], acc_sh.at[dsts[q]], semss[s]).wait()

        def scores(q):
            for g in range(CH // L):
                si = srcs[q][pl.ds(g * L, L)]
                di = dsts[q][pl.ds(g * L, L)]
                e = plsc.load_gather(asl, [si]) + plsc.load_gather(adl, [di])
                e = jnp.where(e >= 0, e, 0.2 * e)
                ex = jnp.exp(e)
                exv[pl.ds(g * L, L)] = ex
                plsc.addupdate_scatter(
                    denv, [lax.shift_right_logical(di, 7),
                           lax.bitwise_and(di, 127)], ex)

        def scale(s):
            def srow(i, carry):
                for jj in range(8):
                    j = i * 8 + jj
                    sv = plsc.load_gather(exv, [jnp.full((L,), j, jnp.int32)])
                    for c in range(F // L):
                        rowss[s][j, pl.ds(c * L, L)] = (
                            rowss[s][j, pl.ds(c * L, L)] * sv)
                return carry
            lax.fori_loop(0, CH // 8, srow, 0)

        for k in range(4):
            idx_start(k, k)
        idx_wait(0, 0)
        idx_wait(1, 1)
        gather_start(0, 0)
        gather_start(1, 1)

        NOCT = NFULL // 8

        def octet(k8, carry):
            for s8 in range(8):
                k = k8 * 8 + s8
                s = s8 % 4
                scores(s8)
                gather_wait(s, s8)
                scale(s)
                scatter_start(s, s8)
                s2 = (s + 2) % 4
                q2 = (s8 + 2) % 8
                q4 = (s8 + 4) % 8

                if s8 < 2:

                    @pl.when(k8 > 0)
                    def _():
                        scatter_wait(s2, q2)
                else:
                    scatter_wait(s2, q2)

                if s8 < 4:
                    idx_start(k + 4, q4)
                    idx_wait(k + 2, q2)
                    gather_start(s2, q2)
                elif s8 < 6:

                    @pl.when(k8 < NOCT - 1)
                    def _():
                        idx_start(k + 4, q4)

                    idx_wait(k + 2, q2)
                    gather_start(s2, q2)
                else:

                    @pl.when(k8 < NOCT - 1)
                    def _():
                        idx_start(k + 4, q4)
                        idx_wait(k + 2, q2)
                        gather_start(s2, q2)
            return carry

        lax.fori_loop(0, NOCT, octet, 0)

        scatter_wait(2, 6)
        scatter_wait(3, 7)

        tbase = ebase + NFULL * CH
        pltpu.sync_copy(src_ref.at[pl.ds(tbase, TAIL)], srcv0.at[pl.ds(0, TAIL)])
        pltpu.sync_copy(dst_ref.at[pl.ds(tbase, TAIL)], dstv0.at[pl.ds(0, TAIL)])
        srcv0[pl.ds(TAIL, L)] = jnp.zeros((L,), jnp.int32)
        dstv0[pl.ds(TAIL, L)] = jnp.full((L,), NPAD - 8, jnp.int32)
        scores(0)
        exv[pl.ds(TAIL, L)] = jnp.zeros((L,), jnp.float32)
        gather_start(0, 0)
        gather_wait(0, 0)
        scale(0)
        scatter_start(0, 0)
        scatter_wait(0, 0)

        plsc.subcore_barrier()

        pltpu.sync_copy(acc_sh.at[pl.ds(sid * RPT, RPT)],
                        acc_out.at[cid, pl.ds(sid * RPT, RPT)])
        pltpu.sync_copy(denv, den_out.at[wid])

    run_branch(src1_ref, dst1_ref, h1_ref, as1_ref, ad1_ref,
               acc1_out, den1_out)
    run_branch(src2_ref, dst2_ref, h2_ref, as2_ref, ad2_ref,
               acc2_out, den2_out)


@functools.lru_cache(maxsize=None)
def _get_sc_edge():
    return functools.partial(
        pl.kernel,
        out_type=[
            jax.ShapeDtypeStruct((NC, NPAD, F), jnp.float32),
            jax.ShapeDtypeStruct((NW, DR, F), jnp.float32),
            jax.ShapeDtypeStruct((NC, NPAD, F), jnp.float32),
            jax.ShapeDtypeStruct((NW, DR, F), jnp.float32),
        ],
        mesh=plsc.VectorSubcoreMesh(core_axis_name="c", subcore_axis_name="s",
                                    num_cores=NC, num_subcores=NS),
        compiler_params=pltpu.CompilerParams(needs_layout_passes=False),
        scratch_types=(
            [
                pltpu.VMEM_SHARED((NPAD, F), jnp.float32),
                pltpu.VMEM((N,), jnp.float32),
                pltpu.VMEM((N,), jnp.float32),
                pltpu.VMEM((DR, F), jnp.float32),
                pltpu.VMEM((CH,), jnp.float32),
            ]
            + [pltpu.VMEM((CH,), jnp.int32) for _ in range(16)]
            + [pltpu.VMEM((CH, F), jnp.float32) for _ in range(4)]
            + [pltpu.SemaphoreType.DMA for _ in range(16)]
        ),
    )(_sc_edge_body)


def _head_body(acc1_ref, den1_ref, acc2_ref, den2_ref, b1_ref, b2_ref,
               batch1_ref, batch2_ref,
               fcW1_ref, fcb1_ref, fcW2_ref, fcb2_ref,
               fcAW_ref, fcAb_ref, fcBW_ref, fcBb_ref, outW_ref, outb_ref,
               out_ref):
    def lr(z):
        return jnp.where(z >= 0, z, 0.01 * z)

    def gat_out(acc_ref, den_ref, b_ref):
        h = acc_ref[0, :N] + acc_ref[1, :N]
        den = jnp.sum(den_ref[...], axis=0)[:N]
        return h / (den[:, None] + 1e-16) + b_ref[...][None, :]

    def pool(x, batch_ref):
        b = batch_ref[...]
        seg = lax.broadcasted_iota(jnp.int32, (G, N), 0)
        P = jnp.where(seg == b[None, :], 1.0, 0.0)
        s = jnp.dot(P, x, preferred_element_type=jnp.float32)
        c = jnp.sum(P, axis=1, keepdims=True)
        return s / jnp.maximum(c, 1.0)

    x = lr(gat_out(acc1_ref, den1_ref, b1_ref))
    x = pool(x, batch1_ref)
    x = lr(jnp.dot(x, fcW1_ref[...], preferred_element_type=jnp.float32)
           + fcb1_ref[...][None, :])

    xt = gat_out(acc2_ref, den2_ref, b2_ref)
    xt = lr(jnp.dot(xt, fcW2_ref[...], preferred_element_type=jnp.float32)
            + fcb2_ref[...][None, :])
    xt = lr(pool(xt, batch2_ref))

    xc = jnp.concatenate([x, xt], axis=1)
    xc = lr(jnp.dot(xc, fcAW_ref[...], preferred_element_type=jnp.float32)
            + fcAb_ref[...][None, :])
    xc = lr(jnp.dot(xc, fcBW_ref[...], preferred_element_type=jnp.float32)
            + fcBb_ref[...][None, :])
    z = (jnp.dot(xc, outW_ref[...], preferred_element_type=jnp.float32)
         + outb_ref[...][None, :])
    out_ref[...] = 1.0 / (1.0 + jnp.exp(-z))


_head = pl.pallas_call(
    _head_body,
    out_shape=jax.ShapeDtypeStruct((G, F), jnp.float32),
)


def kernel(pro1_x, pro1_edge_index, pro1_batch, pro2_x, pro2_edge_index,
           pro2_batch, W1, asrc1, adst1, b1, fcW_p1, fcb_p1,
           W2, asrc2, adst2, b2, fcW_p2, fcb_p2,
           fcAW, fcAb, fcBW, fcBb, outW, outb):
    h1, avec1, h2, avec2 = _prep(
        pro1_x, W1, asrc1, adst1, pro2_x, W2, asrc2, adst2)
    sc_edge = _get_sc_edge()
    acc1, den1, acc2, den2 = sc_edge(
        pro1_edge_index[0], pro1_edge_index[1], h1, avec1[0], avec1[1],
        pro2_edge_index[0], pro2_edge_index[1], h2, avec2[0], avec2[1])
    den1 = den1.reshape(NW, DR * F)
    den2 = den2.reshape(NW, DR * F)
    outWp = jnp.pad(outW, ((0, 0), (0, F - 1)))
    outbp = jnp.pad(outb, ((0, F - 1),))
    o = _head(acc1, den1, acc2, den2, b1, b2, pro1_batch, pro2_batch,
              fcW_p1, fcb_p1, fcW_p2, fcb_p2,
              fcAW, fcAb, fcBW, fcBb, outWp, outbp)
    return o[:, :1]


# speedup vs baseline: 1.2244x; 1.1582x over previous
import functools

import jax
import jax.numpy as jnp
from jax import lax
from jax.experimental import pallas as pl
from jax.experimental.pallas import tpu as pltpu
from jax.experimental.pallas import tpu_sc as plsc

N = 10000
E = 320000
F = 128
G = 16
NC, NS, L = 2, 16, 16
NW = NC * NS
EPW = E // NW
NPAD = 10112
RPT = NPAD // NS
DR = 80
CH = 32
NFULL = 312
TAIL = EPW - NFULL * CH


def _prep_body(x1_ref, W1_ref, as1_ref, ad1_ref, x2_ref, W2_ref, as2_ref,
               ad2_ref, h1_ref, avec1_ref, h2_ref, avec2_ref):
    for x_ref, W_ref, asr, adr, h_ref, avec_ref in (
        (x1_ref, W1_ref, as1_ref, ad1_ref, h1_ref, avec1_ref),
        (x2_ref, W2_ref, as2_ref, ad2_ref, h2_ref, avec2_ref),
    ):
        h = jnp.dot(x_ref[...], W_ref[...], preferred_element_type=jnp.float32)
        h_ref[...] = h
        a_s = jnp.sum(h * asr[...][None, :], axis=1)
        a_d = jnp.sum(h * adr[...][None, :], axis=1)
        avec_ref[...] = jnp.stack([a_s, a_d])


_prep = pl.pallas_call(
    _prep_body,
    out_shape=[
        jax.ShapeDtypeStruct((N, F), jnp.float32),
        jax.ShapeDtypeStruct((2, N), jnp.float32),
        jax.ShapeDtypeStruct((N, F), jnp.float32),
        jax.ShapeDtypeStruct((2, N), jnp.float32),
    ],
)


def _sc_edge_body(src1_ref, dst1_ref, h1_ref, as1_ref, ad1_ref,
                  src2_ref, dst2_ref, h2_ref, as2_ref, ad2_ref,
                  acc1_out, den1_out, acc2_out, den2_out,
                  acc_sh, asl, adl, denv, exv,
                  srcv0, srcv1, srcv2, srcv3, srcv4, srcv5, srcv6, srcv7,
                  dstv0, dstv1, dstv2, dstv3, dstv4, dstv5, dstv6, dstv7,
                  rows0, rows1, rows2, rows3,
                  semg0, semg1, semg2, semg3, sems0, sems1, sems2, sems3,
                  semi0, semi1, semi2, semi3, semi4, semi5, semi6, semi7):
    cid = lax.axis_index("c")
    sid = lax.axis_index("s")
    wid = cid * NS + sid

    def run_branch(src_ref, dst_ref, h_ref, as_ref, ad_ref, acc_out, den_out):
        def zden(r, carry):
            for c in range(F // L):
                denv[r, pl.ds(c * L, L)] = jnp.zeros((L,), jnp.float32)
            return carry
        lax.fori_loop(0, DR, zden, 0)
        for i in range(RPT // DR):
            pltpu.sync_copy(denv, acc_sh.at[pl.ds(sid * RPT + i * DR, DR)])
        rem = RPT % DR
        if rem:
            pltpu.sync_copy(
                denv.at[pl.ds(0, rem)],
                acc_sh.at[pl.ds(sid * RPT + (RPT // DR) * DR, rem)])

        pltpu.sync_copy(as_ref, asl)
        pltpu.sync_copy(ad_ref, adl)

        plsc.subcore_barrier()

        ebase = wid * EPW
        srcs = [srcv0, srcv1, srcv2, srcv3, srcv4, srcv5, srcv6, srcv7]
        dsts = [dstv0, dstv1, dstv2, dstv3, dstv4, dstv5, dstv6, dstv7]
        rowss = [rows0, rows1, rows2, rows3]
        semgs = [semg0, semg1, semg2, semg3]
        semss = [sems0, sems1, sems2, sems3]
        semis = [semi0, semi1, semi2, semi3, semi4, semi5, semi6, semi7]

        def idx_start(k, q):
            base = ebase + k * CH
            pltpu.async_copy(src_ref.at[pl.ds(base, CH)], srcs[q], semis[q])
            pltpu.async_copy(dst_ref.at[pl.ds(base, CH)], dsts[q], semis[q])

        def idx_wait(k, q):
            base = ebase + k * CH
            pltpu.make_async_copy(src_ref.at[pl.ds(base, CH)], srcs[q],
                                  semis[q]).wait()
            pltpu.make_async_copy(dst_ref.at[pl.ds(base, CH)], dsts[q],
                                  semis[q]).wait()

        def gather_start(s, q):
            pltpu.async_copy(h_ref.at[srcs[q]], rowss[s], semgs[s])

        def gather_wait(s, q):
            pltpu.make_async_copy(h_ref.at[srcs[q]], rowss[s], semgs[s]).wait()

        def scatter_start(s, q):
            pltpu.async_copy(rowss[s], acc_sh.at[dsts[q]], semss[s], add=True)

        def scatter_wait(s, q):
            pltpu.make_async_copy(rowss[s], acc_sh.at[dsts[q]], semss[s]).wait()

        def scores(q):
            for g in range(CH // L):
                si = srcs[q][pl.ds(g * L, L)]
                di = dsts[q][pl.ds(g * L, L)]
                e = plsc.load_gather(asl, [si]) + plsc.load_gather(adl, [di])
                e = jnp.where(e >= 0, e, 0.2 * e)
                ex = jnp.exp(e)
                exv[pl.ds(g * L, L)] = ex
                plsc.addupdate_scatter(
                    denv, [lax.shift_right_logical(di, 7),
                           lax.bitwise_and(di, 127)], ex)

        def scale(s):
            def srow(i, carry):
                for jj in range(8):
                    j = i * 8 + jj
                    sv = plsc.load_gather(exv, [jnp.full((L,), j, jnp.int32)])
                    for c in range(F // L):
                        rowss[s][j, pl.ds(c * L, L)] = (
                            rowss[s][j, pl.ds(c * L, L)] * sv)
                return carry
            lax.fori_loop(0, CH // 8, srow, 0)

        for k in range(4):
            idx_start(k, k)
        idx_wait(0, 0)
        idx_wait(1, 1)
        gather_start(0, 0)
        gather_start(1, 1)

        NOCT = NFULL // 8

        def octet(k8, carry):
            for s8 in range(8):
                k = k8 * 8 + s8
                s = s8 % 4
                s2 = (s + 2) % 4
                q2 = (s8 + 2) % 8
                q4 = (s8 + 4) % 8
                scores(s8)
                gather_wait(s, s8)

                if s8 < 2:

                    @pl.when(k8 > 0)
                    def _():
                        scatter_wait(s2, q2)
                else:
                    scatter_wait(s2, q2)

                if s8 < 4:
                    idx_start(k + 4, q4)
                    idx_wait(k + 2, q2)
                    gather_start(s2, q2)
                elif s8 < 6:

                    @pl.when(k8 < NOCT - 1)
                    def _():
                        idx_start(k + 4, q4)

                    idx_wait(k + 2, q2)
                    gather_start(s2, q2)
                else:

                    @pl.when(k8 < NOCT - 1)
                    def _():
                        idx_start(k + 4, q4)
                        idx_wait(k + 2, q2)
                        gather_start(s2, q2)

                scale(s)
                scatter_start(s, s8)
            return carry

        lax.fori_loop(0, NOCT, octet, 0)

        scatter_wait(2, 6)
        scatter_wait(3, 7)

        tbase = ebase + NFULL * CH
        pltpu.sync_copy(src_ref.at[pl.ds(tbase, TAIL)], srcv0.at[pl.ds(0, TAIL)])
        pltpu.sync_copy(dst_ref.at[pl.ds(tbase, TAIL)], dstv0.at[pl.ds(0, TAIL)])
        srcv0[pl.ds(TAIL, L)] = jnp.zeros((L,), jnp.int32)
        dstv0[pl.ds(TAIL, L)] = jnp.full((L,), NPAD - 8, jnp.int32)
        scores(0)
        exv[pl.ds(TAIL, L)] = jnp.zeros((L,), jnp.float32)
        gather_start(0, 0)
        gather_wait(0, 0)
        scale(0)
        scatter_start(0, 0)
        scatter_wait(0, 0)

        plsc.subcore_barrier()

        pltpu.sync_copy(acc_sh.at[pl.ds(sid * RPT, RPT)],
                        acc_out.at[cid, pl.ds(sid * RPT, RPT)])
        pltpu.sync_copy(denv, den_out.at[wid])

    run_branch(src1_ref, dst1_ref, h1_ref, as1_ref, ad1_ref,
               acc1_out, den1_out)
    run_branch(src2_ref, dst2_ref, h2_ref, as2_ref, ad2_ref,
               acc2_out, den2_out)


@functools.lru_cache(maxsize=None)
def _get_sc_edge():
    return functools.partial(
        pl.kernel,
        out_type=[
            jax.ShapeDtypeStruct((NC, NPAD, F), jnp.float32),
            jax.ShapeDtypeStruct((NW, DR, F), jnp.float32),
            jax.ShapeDtypeStruct((NC, NPAD, F), jnp.float32),
            jax.ShapeDtypeStruct((NW, DR, F), jnp.float32),
        ],
        mesh=plsc.VectorSubcoreMesh(core_axis_name="c", subcore_axis_name="s",
                                    num_cores=NC, num_subcores=NS),
        compiler_params=pltpu.CompilerParams(needs_layout_passes=False),
        scratch_types=(
            [
                pltpu.VMEM_SHARED((NPAD, F), jnp.float32),
                pltpu.VMEM((N,), jnp.float32),
                pltpu.VMEM((N,), jnp.float32),
                pltpu.VMEM((DR, F), jnp.float32),
                pltpu.VMEM((CH,), jnp.float32),
            ]
            + [pltpu.VMEM((CH,), jnp.int32) for _ in range(16)]
            + [pltpu.VMEM((CH, F), jnp.float32) for _ in range(4)]
            + [pltpu.SemaphoreType.DMA for _ in range(16)]
        ),
    )(_sc_edge_body)


def _head_body(acc1_ref, den1_ref, acc2_ref, den2_ref, b1_ref, b2_ref,
               batch1_ref, batch2_ref,
               fcW1_ref, fcb1_ref, fcW2_ref, fcb2_ref,
               fcAW_ref, fcAb_ref, fcBW_ref, fcBb_ref, outW_ref, outb_ref,
               out_ref):
    def lr(z):
        return jnp.where(z >= 0, z, 0.01 * z)

    def gat_out(acc_ref, den_ref, b_ref):
        h = acc_ref[0, :N] + acc_ref[1, :N]
        den = jnp.sum(den_ref[...], axis=0)[:N]
        return h / (den[:, None] + 1e-16) + b_ref[...][None, :]

    def pool(x, batch_ref):
        b = batch_ref[...]
        seg = lax.broadcasted_iota(jnp.int32, (G, N), 0)
        P = jnp.where(seg == b[None, :], 1.0, 0.0)
        s = jnp.dot(P, x, preferred_element_type=jnp.float32)
        c = jnp.sum(P, axis=1, keepdims=True)
        return s / jnp.maximum(c, 1.0)

    x = lr(gat_out(acc1_ref, den1_ref, b1_ref))
    x = pool(x, batch1_ref)
    x = lr(jnp.dot(x, fcW1_ref[...], preferred_element_type=jnp.float32)
           + fcb1_ref[...][None, :])

    xt = gat_out(acc2_ref, den2_ref, b2_ref)
    xt = lr(jnp.dot(xt, fcW2_ref[...], preferred_element_type=jnp.float32)
            + fcb2_ref[...][None, :])
    xt = lr(pool(xt, batch2_ref))

    xc = jnp.concatenate([x, xt], axis=1)
    xc = lr(jnp.dot(xc, fcAW_ref[...], preferred_element_type=jnp.float32)
            + fcAb_ref[...][None, :])
    xc = lr(jnp.dot(xc, fcBW_ref[...], preferred_element_type=jnp.float32)
            + fcBb_ref[...][None, :])
    z = (jnp.dot(xc, outW_ref[...], preferred_element_type=jnp.float32)
         + outb_ref[...][None, :])
    out_ref[...] = 1.0 / (1.0 + jnp.exp(-z))


_head = pl.pallas_call(
    _head_body,
    out_shape=jax.ShapeDtypeStruct((G, F), jnp.float32),
)


def kernel(pro1_x, pro1_edge_index, pro1_batch, pro2_x, pro2_edge_index,
           pro2_batch, W1, asrc1, adst1, b1, fcW_p1, fcb_p1,
           W2, asrc2, adst2, b2, fcW_p2, fcb_p2,
           fcAW, fcAb, fcBW, fcBb, outW, outb):
    h1, avec1, h2, avec2 = _prep(
        pro1_x, W1, asrc1, adst1, pro2_x, W2, asrc2, adst2)
    sc_edge = _get_sc_edge()
    acc1, den1, acc2, den2 = sc_edge(
        pro1_edge_index[0], pro1_edge_index[1], h1, avec1[0], avec1[1],
        pro2_edge_index[0], pro2_edge_index[1], h2, avec2[0], avec2[1])
    den1 = den1.reshape(NW, DR * F)
    den2 = den2.reshape(NW, DR * F)
    outWp = jnp.pad(outW, ((0, 0), (0, F - 1)))
    outbp = jnp.pad(outb, ((0, F - 1),))
    o = _head(acc1, den1, acc2, den2, b1, b2, pro1_batch, pro2_batch,
              fcW_p1, fcb_p1, fcW_p2, fcb_p2,
              fcAW, fcAb, fcBW, fcBb, outWp, outbp)
    return o[:, :1]
